# baseline (device time: 111340 ns/iter reference)
import functools

import jax
import jax.numpy as jnp
from jax import lax
from jax.experimental import pallas as pl
from jax.experimental.pallas import tpu as pltpu

B, S, D = 1, 1024, 2048
H, DH, DR = 16, 128, 32
DC_SH = 128
SCALE = (DH + DR) ** -0.5
LOG2E = 1.4426950408889634


def _proj_body(
    x_ref, wdkv_ref, wuk_ref, wuv_ref, wq_ref, wqr_ref, wkr_ref,
    q_ref, qr_ref, kr_ref, k_ref, v_ref,
    c_loc, c_peer, wuk_peer, wuv_peer, send_sems, recv_sems,
):
    mx = lax.axis_index("x")
    my = lax.axis_index("y")
    mz = lax.axis_index("z")
    peer = (1 - mx, my, mz)

    barrier = pltpu.get_barrier_semaphore()
    pl.semaphore_signal(
        barrier, inc=1, device_id=peer, device_id_type=pl.DeviceIdType.MESH
    )
    pl.semaphore_wait(barrier, 1)

    xb = x_ref[...]
    c_loc[...] = jnp.dot(
        xb, wdkv_ref[...], preferred_element_type=jnp.float32
    ).astype(jnp.bfloat16)

    rdmas = []
    for i, (src, dst) in enumerate(
        [(c_loc, c_peer), (wuk_ref, wuk_peer), (wuv_ref, wuv_peer)]
    ):
        r = pltpu.make_async_remote_copy(
            src_ref=src,
            dst_ref=dst,
            send_sem=send_sems.at[i],
            recv_sem=recv_sems.at[i],
            device_id=peer,
            device_id_type=pl.DeviceIdType.MESH,
        )
        r.start()
        rdmas.append(r)

    q_ref[...] = (
        jnp.dot(xb, wq_ref[...], preferred_element_type=jnp.float32)
        * (SCALE * LOG2E)
    ).astype(jnp.bfloat16)
    qr_ref[...] = (
        jnp.dot(xb, wqr_ref[...], preferred_element_type=jnp.float32)
        * (SCALE * LOG2E)
    ).astype(jnp.bfloat16)
    kr_ref[...] = jnp.dot(
        xb, wkr_ref[...], preferred_element_type=jnp.float32
    ).astype(jnp.bfloat16)

    for r in rdmas:
        r.wait()

    k_ref[...] = (
        jnp.dot(c_loc[...], wuk_ref[...], preferred_element_type=jnp.float32)
        + jnp.dot(c_peer[...], wuk_peer[...], preferred_element_type=jnp.float32)
    ).astype(jnp.bfloat16)
    v_ref[...] = (
        jnp.dot(c_loc[...], wuv_ref[...], preferred_element_type=jnp.float32)
        + jnp.dot(c_peer[...], wuv_peer[...], preferred_element_type=jnp.float32)
    ).astype(jnp.bfloat16)


def _attn_body(q_ref, k_ref, v_ref, qr_ref, kr_ref, o_ref):
    s = lax.dot_general(
        q_ref[...], k_ref[...],
        (((1,), (1,)), ((), ())),
        preferred_element_type=jnp.float32,
    )
    s = s + lax.dot_general(
        qr_ref[0], kr_ref[...],
        (((1,), (1,)), ((), ())),
        preferred_element_type=jnp.float32,
    )
    p = jnp.exp2(s.astype(jnp.bfloat16))
    ones = jnp.ones((S, DH), jnp.bfloat16)
    rowsum = lax.dot_general(
        p, ones, (((1,), (0,)), ((), ())),
        preferred_element_type=jnp.float32,
    )[:, :1]
    o = lax.dot_general(
        p, v_ref[...],
        (((1,), (0,)), ((), ())),
        preferred_element_type=jnp.float32,
    )
    o_ref[...] = (o * (1.0 / rowsum)).astype(jnp.bfloat16)


def _outproj_body(o_ref, wo_ref, out_ref):
    out_ref[...] = jnp.dot(
        o_ref[...], wo_ref[...], preferred_element_type=jnp.float32
    )


def kernel(x, Wdkv, Wuk, Wuv, Wq, Wqr, Wkr, Wo):
    bf = jnp.bfloat16
    xb = x.reshape(S, D).astype(bf)
    wdkv = Wdkv.astype(bf)
    wuk = Wuk.astype(bf)
    wuv = Wuv.astype(bf)
    wq = Wq.astype(bf)
    wqr = Wqr.astype(bf)
    wkr = Wkr.astype(bf)
    wo = Wo.astype(bf)

    q, qr, kr, k, v = pl.pallas_call(
        _proj_body,
        out_shape=[
            jax.ShapeDtypeStruct((S, D), bf),
            jax.ShapeDtypeStruct((S, H * DR), bf),
            jax.ShapeDtypeStruct((S, DR), bf),
            jax.ShapeDtypeStruct((S, D), bf),
            jax.ShapeDtypeStruct((S, D), bf),
        ],
        in_specs=[pl.BlockSpec(memory_space=pltpu.VMEM)] * 7,
        out_specs=[pl.BlockSpec(memory_space=pltpu.VMEM)] * 5,
        scratch_shapes=[
            pltpu.VMEM((S, DC_SH), bf),
            pltpu.VMEM((S, DC_SH), bf),
            pltpu.VMEM((DC_SH, D), bf),
            pltpu.VMEM((DC_SH, D), bf),
            pltpu.SemaphoreType.DMA((3,)),
            pltpu.SemaphoreType.DMA((3,)),
        ],
        compiler_params=pltpu.CompilerParams(collective_id=0),
    )(xb, wdkv, wuk, wuv, wq, wqr, wkr)

    qr3 = qr.reshape(S, H, DR).transpose(1, 0, 2)

    o = pl.pallas_call(
        _attn_body,
        grid=(H,),
        out_shape=jax.ShapeDtypeStruct((S, D), bf),
        in_specs=[
            pl.BlockSpec((S, DH), lambda h: (0, h)),
            pl.BlockSpec((S, DH), lambda h: (0, h)),
            pl.BlockSpec((S, DH), lambda h: (0, h)),
            pl.BlockSpec((1, S, DR), lambda h: (h, 0, 0)),
            pl.BlockSpec((S, DR), lambda h: (0, 0)),
        ],
        out_specs=pl.BlockSpec((S, DH), lambda h: (0, h)),
        compiler_params=pltpu.CompilerParams(
            dimension_semantics=("arbitrary",)
        ),
    )(q, k, v, qr3, kr)

    NJ = 4
    out = pl.pallas_call(
        _outproj_body,
        grid=(NJ,),
        out_shape=jax.ShapeDtypeStruct((S, D), jnp.float32),
        in_specs=[
            pl.BlockSpec((S, D), lambda j: (0, 0)),
            pl.BlockSpec((D, D // NJ), lambda j: (0, j)),
        ],
        out_specs=pl.BlockSpec((S, D // NJ), lambda j: (0, j)),
        compiler_params=pltpu.CompilerParams(
            dimension_semantics=("arbitrary",)
        ),
    )(o, wo)

    return out.reshape(B, S, D)


# device time: 98937 ns/iter; 1.1254x vs baseline; 1.1254x over previous
import functools

import jax
import jax.numpy as jnp
from jax import lax
from jax.experimental import pallas as pl
from jax.experimental.pallas import tpu as pltpu

B, S, D = 1, 1024, 2048
H, DH, DR = 16, 128, 32
DC_SH = 128
SCALE = (DH + DR) ** -0.5
LOG2E = 1.4426950408889634


def _proj_body(
    x_ref, wdkv_ref, wuk_ref, wuv_ref, wq_ref, wqr_ref, wkr_ref,
    q_ref, qr_ref, kr_ref, k_ref, v_ref,
    c_loc, c_peer, wuk_peer, wuv_peer, send_sems, recv_sems,
):
    mx = lax.axis_index("x")
    my = lax.axis_index("y")
    mz = lax.axis_index("z")
    peer = (1 - mx, my, mz)

    barrier = pltpu.get_barrier_semaphore()
    pl.semaphore_signal(
        barrier, inc=1, device_id=peer, device_id_type=pl.DeviceIdType.MESH
    )
    pl.semaphore_wait(barrier, 1)

    xb = x_ref[...]
    c_loc[...] = jnp.dot(
        xb, wdkv_ref[...], preferred_element_type=jnp.float32
    ).astype(jnp.bfloat16)

    rdmas = []
    for i, (src, dst) in enumerate(
        [(c_loc, c_peer), (wuk_ref, wuk_peer), (wuv_ref, wuv_peer)]
    ):
        r = pltpu.make_async_remote_copy(
            src_ref=src,
            dst_ref=dst,
            send_sem=send_sems.at[i],
            recv_sem=recv_sems.at[i],
            device_id=peer,
            device_id_type=pl.DeviceIdType.MESH,
        )
        r.start()
        rdmas.append(r)

    q_ref[...] = (
        jnp.dot(xb, wq_ref[...], preferred_element_type=jnp.float32)
        * (SCALE * LOG2E)
    ).astype(jnp.bfloat16)
    qr_ref[...] = (
        jnp.dot(xb, wqr_ref[...], preferred_element_type=jnp.float32)
        * (SCALE * LOG2E)
    ).astype(jnp.bfloat16)
    kr_ref[...] = jnp.dot(
        xb, wkr_ref[...], preferred_element_type=jnp.float32
    ).astype(jnp.bfloat16)

    for r in rdmas:
        r.wait()

    k_ref[...] = (
        jnp.dot(c_loc[...], wuk_ref[...], preferred_element_type=jnp.float32)
        + jnp.dot(c_peer[...], wuk_peer[...], preferred_element_type=jnp.float32)
    ).astype(jnp.bfloat16)
    v_ref[...] = (
        jnp.dot(c_loc[...], wuv_ref[...], preferred_element_type=jnp.float32)
        + jnp.dot(c_peer[...], wuv_peer[...], preferred_element_type=jnp.float32)
    ).astype(jnp.bfloat16)


HPS = 2


def _attn_body(q_ref, k_ref, v_ref, qr_ref, kr_ref, o_ref):
    for i in range(HPS):
        hs = slice(i * DH, (i + 1) * DH)
        s = lax.dot_general(
            q_ref[:, hs], k_ref[:, hs],
            (((1,), (1,)), ((), ())),
            preferred_element_type=jnp.float32,
        )
        s = s + lax.dot_general(
            qr_ref[i], kr_ref[...],
            (((1,), (1,)), ((), ())),
            preferred_element_type=jnp.float32,
        )
        p = jnp.exp2(s)
        recip = 1.0 / jnp.sum(p, axis=1, keepdims=True)
        o = lax.dot_general(
            p.astype(jnp.bfloat16), v_ref[:, hs],
            (((1,), (0,)), ((), ())),
            preferred_element_type=jnp.float32,
        )
        o_ref[:, hs] = (o * recip).astype(jnp.bfloat16)


def _outproj_body(o_ref, wo_ref, out_ref):
    out_ref[...] = jnp.dot(
        o_ref[...], wo_ref[...], preferred_element_type=jnp.float32
    )


def kernel(x, Wdkv, Wuk, Wuv, Wq, Wqr, Wkr, Wo):
    bf = jnp.bfloat16
    xb = x.reshape(S, D).astype(bf)
    wdkv = Wdkv.astype(bf)
    wuk = Wuk.astype(bf)
    wuv = Wuv.astype(bf)
    wq = Wq.astype(bf)
    wqr = Wqr.astype(bf)
    wkr = Wkr.astype(bf)
    wo = Wo.astype(bf)

    q, qr, kr, k, v = pl.pallas_call(
        _proj_body,
        out_shape=[
            jax.ShapeDtypeStruct((S, D), bf),
            jax.ShapeDtypeStruct((S, H * DR), bf),
            jax.ShapeDtypeStruct((S, DR), bf),
            jax.ShapeDtypeStruct((S, D), bf),
            jax.ShapeDtypeStruct((S, D), bf),
        ],
        in_specs=[pl.BlockSpec(memory_space=pltpu.VMEM)] * 7,
        out_specs=[pl.BlockSpec(memory_space=pltpu.VMEM)] * 5,
        scratch_shapes=[
            pltpu.VMEM((S, DC_SH), bf),
            pltpu.VMEM((S, DC_SH), bf),
            pltpu.VMEM((DC_SH, D), bf),
            pltpu.VMEM((DC_SH, D), bf),
            pltpu.SemaphoreType.DMA((3,)),
            pltpu.SemaphoreType.DMA((3,)),
        ],
        compiler_params=pltpu.CompilerParams(collective_id=0),
    )(xb, wdkv, wuk, wuv, wq, wqr, wkr)

    qr3 = qr.reshape(S, H, DR).transpose(1, 0, 2)

    o = pl.pallas_call(
        _attn_body,
        grid=(H // HPS,),
        out_shape=jax.ShapeDtypeStruct((S, D), bf),
        in_specs=[
            pl.BlockSpec((S, HPS * DH), lambda h: (0, h)),
            pl.BlockSpec((S, HPS * DH), lambda h: (0, h)),
            pl.BlockSpec((S, HPS * DH), lambda h: (0, h)),
            pl.BlockSpec((HPS, S, DR), lambda h: (h, 0, 0)),
            pl.BlockSpec((S, DR), lambda h: (0, 0)),
        ],
        out_specs=pl.BlockSpec((S, HPS * DH), lambda h: (0, h)),
        compiler_params=pltpu.CompilerParams(
            dimension_semantics=("arbitrary",)
        ),
    )(q, k, v, qr3, kr)

    NJ = 4
    out = pl.pallas_call(
        _outproj_body,
        grid=(NJ,),
        out_shape=jax.ShapeDtypeStruct((S, D), jnp.float32),
        in_specs=[
            pl.BlockSpec((S, D), lambda j: (0, 0)),
            pl.BlockSpec((D, D // NJ), lambda j: (0, j)),
        ],
        out_specs=pl.BlockSpec((S, D // NJ), lambda j: (0, j)),
        compiler_params=pltpu.CompilerParams(
            dimension_semantics=("arbitrary",)
        ),
    )(o, wo)

    return out.reshape(B, S, D)


# device time: 80890 ns/iter; 1.3764x vs baseline; 1.2231x over previous
import functools

import jax
import jax.numpy as jnp
from jax import lax
from jax.experimental import pallas as pl
from jax.experimental.pallas import tpu as pltpu

B, S, D = 1, 1024, 2048
H, DH, DR = 16, 128, 32
DC_SH = 128
SCALE = (DH + DR) ** -0.5
LOG2E = 1.4426950408889634


NQJ = 4


def _proj_rdmas(c_loc, c_peer, wuk_ref, wuk_peer, wuv_ref, wuv_peer,
                send_sems, recv_sems, peer):
    return [
        pltpu.make_async_remote_copy(
            src_ref=src,
            dst_ref=dst,
            send_sem=send_sems.at[i],
            recv_sem=recv_sems.at[i],
            device_id=peer,
            device_id_type=pl.DeviceIdType.MESH,
        )
        for i, (src, dst) in enumerate(
            [(c_loc, c_peer), (wuk_ref, wuk_peer), (wuv_ref, wuv_peer)]
        )
    ]


def _proj_body(
    x_ref, wdkv_ref, wuk_ref, wuv_ref, wq_ref, wqr_ref, wkr_ref,
    q_ref, qr_ref, kr_ref, k_ref, v_ref,
    xb, c_loc, c_peer, wuk_peer, wuv_peer, send_sems, recv_sems,
):
    j = pl.program_id(0)
    mx = lax.axis_index("x")
    my = lax.axis_index("y")
    mz = lax.axis_index("z")
    peer = (1 - mx, my, mz)

    @pl.when(j == 0)
    def _():
        barrier = pltpu.get_barrier_semaphore()
        pl.semaphore_signal(
            barrier, inc=1, device_id=peer,
            device_id_type=pl.DeviceIdType.MESH,
        )
        pl.semaphore_wait(barrier, 1)

        xb[...] = x_ref[...].astype(jnp.bfloat16)
        c_loc[...] = jnp.dot(
            xb[...], wdkv_ref[...], preferred_element_type=jnp.float32
        ).astype(jnp.bfloat16)

        for r in _proj_rdmas(c_loc, c_peer, wuk_ref, wuk_peer, wuv_ref,
                             wuv_peer, send_sems, recv_sems, peer):
            r.start()

        qr_ref[...] = (
            jnp.dot(xb[...], wqr_ref[...], preferred_element_type=jnp.float32)
            * (SCALE * LOG2E)
        ).astype(jnp.bfloat16)
        kr_ref[...] = jnp.dot(
            xb[...], wkr_ref[...], preferred_element_type=jnp.float32
        ).astype(jnp.bfloat16)

    q_ref[...] = (
        jnp.dot(xb[...], wq_ref[...].astype(jnp.bfloat16),
                preferred_element_type=jnp.float32)
        * (SCALE * LOG2E)
    ).astype(jnp.bfloat16)

    @pl.when(j == NQJ - 1)
    def _():
        for r in _proj_rdmas(c_loc, c_peer, wuk_ref, wuk_peer, wuv_ref,
                             wuv_peer, send_sems, recv_sems, peer):
            r.wait()
        k_ref[...] = (
            jnp.dot(c_loc[...], wuk_ref[...],
                    preferred_element_type=jnp.float32)
            + jnp.dot(c_peer[...], wuk_peer[...],
                      preferred_element_type=jnp.float32)
        ).astype(jnp.bfloat16)
        v_ref[...] = (
            jnp.dot(c_loc[...], wuv_ref[...],
                    preferred_element_type=jnp.float32)
            + jnp.dot(c_peer[...], wuv_peer[...],
                      preferred_element_type=jnp.float32)
        ).astype(jnp.bfloat16)


HPS = 8


def _attn_body(q_ref, k_ref, v_ref, qr_ref, kr_ref, o_ref):
    for i in range(HPS):
        hs = slice(i * DH, (i + 1) * DH)
        s = lax.dot_general(
            q_ref[:, hs], k_ref[:, hs],
            (((1,), (1,)), ((), ())),
            preferred_element_type=jnp.float32,
        )
        s = s + lax.dot_general(
            qr_ref[:, i * DR:(i + 1) * DR], kr_ref[...],
            (((1,), (1,)), ((), ())),
            preferred_element_type=jnp.float32,
        )
        p = jnp.exp2(s)
        recip = 1.0 / jnp.sum(p, axis=1, keepdims=True)
        o = lax.dot_general(
            p.astype(jnp.bfloat16), v_ref[:, hs],
            (((1,), (0,)), ((), ())),
            preferred_element_type=jnp.float32,
        )
        o_ref[:, hs] = (o * recip).astype(jnp.bfloat16)


def _outproj_body(o_ref, wo_ref, out_ref):
    out_ref[...] = jnp.dot(
        o_ref[...], wo_ref[...], preferred_element_type=jnp.float32
    ).astype(jnp.bfloat16)


def kernel(x, Wdkv, Wuk, Wuv, Wq, Wqr, Wkr, Wo):
    bf = jnp.bfloat16
    xf = x.reshape(S, D)
    wdkv = Wdkv.astype(bf)
    wuk = Wuk.astype(bf)
    wuv = Wuv.astype(bf)
    wqr = Wqr.astype(bf)
    wkr = Wkr.astype(bf)
    wo = Wo.astype(bf)

    DQ = D // NQJ
    q, qr, kr, k, v = pl.pallas_call(
        _proj_body,
        grid=(NQJ,),
        out_shape=[
            jax.ShapeDtypeStruct((S, D), bf),
            jax.ShapeDtypeStruct((S, H * DR), bf),
            jax.ShapeDtypeStruct((S, DR), bf),
            jax.ShapeDtypeStruct((S, D), bf),
            jax.ShapeDtypeStruct((S, D), bf),
        ],
        in_specs=[
            pl.BlockSpec((S, D), lambda j: (0, 0)),
            pl.BlockSpec((D, DC_SH), lambda j: (0, 0)),
            pl.BlockSpec((DC_SH, D), lambda j: (0, 0)),
            pl.BlockSpec((DC_SH, D), lambda j: (0, 0)),
            pl.BlockSpec((D, DQ), lambda j: (0, j)),
            pl.BlockSpec((D, H * DR), lambda j: (0, 0)),
            pl.BlockSpec((D, DR), lambda j: (0, 0)),
        ],
        out_specs=[
            pl.BlockSpec((S, DQ), lambda j: (0, j)),
            pl.BlockSpec((S, H * DR), lambda j: (0, 0)),
            pl.BlockSpec((S, DR), lambda j: (0, 0)),
            pl.BlockSpec((S, D), lambda j: (0, 0)),
            pl.BlockSpec((S, D), lambda j: (0, 0)),
        ],
        scratch_shapes=[
            pltpu.VMEM((S, D), bf),
            pltpu.VMEM((S, DC_SH), bf),
            pltpu.VMEM((S, DC_SH), bf),
            pltpu.VMEM((DC_SH, D), bf),
            pltpu.VMEM((DC_SH, D), bf),
            pltpu.SemaphoreType.DMA((3,)),
            pltpu.SemaphoreType.DMA((3,)),
        ],
        compiler_params=pltpu.CompilerParams(
            collective_id=0,
            dimension_semantics=("arbitrary",),
        ),
    )(xf, wdkv, wuk, wuv, Wq, wqr, wkr)

    o = pl.pallas_call(
        _attn_body,
        grid=(H // HPS,),
        out_shape=jax.ShapeDtypeStruct((S, D), bf),
        in_specs=[
            pl.BlockSpec((S, HPS * DH), lambda h: (0, h)),
            pl.BlockSpec((S, HPS * DH), lambda h: (0, h)),
            pl.BlockSpec((S, HPS * DH), lambda h: (0, h)),
            pl.BlockSpec((S, HPS * DR), lambda h: (0, h)),
            pl.BlockSpec((S, DR), lambda h: (0, 0)),
        ],
        out_specs=pl.BlockSpec((S, HPS * DH), lambda h: (0, h)),
        compiler_params=pltpu.CompilerParams(
            dimension_semantics=("arbitrary",)
        ),
    )(q, k, v, qr, kr)

    NJ = 4
    out = pl.pallas_call(
        _outproj_body,
        grid=(NJ,),
        out_shape=jax.ShapeDtypeStruct((S, D), bf),
        in_specs=[
            pl.BlockSpec((S, D), lambda j: (0, 0)),
            pl.BlockSpec((D, D // NJ), lambda j: (0, j)),
        ],
        out_specs=pl.BlockSpec((S, D // NJ), lambda j: (0, j)),
        compiler_params=pltpu.CompilerParams(
            dimension_semantics=("arbitrary",)
        ),
    )(o, wo)

    return out.reshape(B, S, D)
